# baseline (speedup 1.0000x reference)
import functools

import jax
import jax.numpy as jnp
from jax import lax
from jax.experimental import pallas as pl
from jax.experimental.pallas import tpu as pltpu
from jax.experimental.pallas import tpu_sc as plsc

_MAX_ITEMS = 1000000
_B = 16384
_D = 128
_L = 16
_NW = 32
_BATCH_ROWS = 128
_PAD = 2 * _BATCH_ROWS
_WEIGHT = 0.1
_MOMENT = 0.5
_WARM_UP = 1.0 / 1000.0
_COOL_DOWN = 1.0 / 100000.0


def _sc_call(activations, samples, hist_ref):
    mesh = plsc.VectorSubcoreMesh(core_axis_name="c", subcore_axis_name="s")

    @functools.partial(
        pl.kernel,
        out_type=(
            jax.ShapeDtypeStruct((_NW, _L), jnp.float32),
            jax.ShapeDtypeStruct((_B + _L, _D), jnp.float32),
        ),
        mesh=mesh,
        compiler_params=pltpu.CompilerParams(needs_layout_passes=False),
        scratch_types=[
            pltpu.VMEM((_B,), jnp.int32),
            pltpu.VMEM((_B + _PAD,), jnp.int32),
            pltpu.VMEM((_B + _PAD,), jnp.int32),
            pltpu.VMEM((_B + _PAD,), jnp.int32),
            pltpu.VMEM((_BATCH_ROWS, _D), jnp.float32),
            pltpu.VMEM((_BATCH_ROWS, _D), jnp.float32),
            pltpu.VMEM((_BATCH_ROWS, _D), jnp.float32),
            pltpu.VMEM((_L, _D), jnp.float32),
            pltpu.VMEM((_L,), jnp.float32),
            pltpu.SemaphoreType.DMA,
            pltpu.SemaphoreType.DMA,
            pltpu.SemaphoreType.DMA,
        ],
    )
    def body(act_hbm, smp_hbm, hist, loss_hbm, upd_hbm,
             smp_v, pos_v, sid_v, posu_v, old_b, act_b, u_b, cur_v, lss_v,
             s0, s1, s2):
        wid = lax.axis_index("s") * 2 + lax.axis_index("c")
        lanes = lax.iota(jnp.int32, _L)

        pltpu.sync_copy(smp_hbm, smp_v)

        def build(v, cnt):
            s = smp_v[pl.ds(v * _L, _L)]
            own = (jnp.bitwise_and(s, _NW - 1) == wid) & (s > 0) & (s < _MAX_ITEMS)
            inc = jnp.cumsum(own.astype(jnp.int32))
            offs = cnt + inc - 1
            plsc.store_scatter(pos_v, [offs], v * _L + lanes, mask=own)
            plsc.store_scatter(sid_v, [offs], s, mask=own)
            plsc.store_scatter(posu_v, [offs], v * _L + lanes, mask=own)
            return cnt + jnp.max(inc)

        n_own = lax.fori_loop(0, _B // _L, build, jnp.int32(0))
        for pc in range(_BATCH_ROWS // _L):
            idxs = n_own + pc * _L + lanes
            plsc.store_scatter(pos_v, [idxs], lanes * 0)
            plsc.store_scatter(sid_v, [idxs], lanes * 0 + _MAX_ITEMS)
            plsc.store_scatter(posu_v, [idxs], _B + lanes)

        nchunks = (n_own + _L - 1) // _L
        nbatch = (n_own + _BATCH_ROWS - 1) // _BATCH_ROWS

        def p1(kb, lossvec):
            b0 = kb * _BATCH_ROWS
            cp_a = pltpu.async_copy(
                act_hbm.at[pos_v.at[pl.ds(b0, _BATCH_ROWS)]], act_b, s0)
            cp_h = pltpu.async_copy(
                hist.at[sid_v.at[pl.ds(b0, _BATCH_ROWS)]], old_b, s1)
            cp_a.wait()
            cp_h.wait()
            descs = []
            for g in range(_BATCH_ROWS // _L):
                valid = (b0 + g * _L + lanes) < n_own
                rows = g * _L + lanes

                def col(c4, lv, rows=rows, valid=valid):
                    for cc in range(4):
                        colv = lanes * 0 + (c4 * 4 + cc)
                        o = plsc.load_gather(old_b, [rows, colv])
                        a = plsc.load_gather(act_b, [rows, colv])
                        d = jnp.where(valid, o - a, 0.0)
                        plsc.store_scatter(u_b, [rows, colv], (_MOMENT - 1.0) * d)
                        lv = lv + d * d
                    return lv

                lossvec = lax.fori_loop(0, _D // 4, col, lossvec)
                posu = posu_v[pl.ds(b0 + g * _L, _L)]
                descs.append(pltpu.async_copy(
                    u_b.at[pl.ds(g * _L, _L)], upd_hbm.at[posu], s2))
            for dsc in descs:
                dsc.wait()
            return lossvec

        lossvec = lax.fori_loop(0, nbatch, p1, jnp.zeros((_L,), jnp.float32))
        lss_v[...] = lossvec
        pltpu.sync_copy(lss_v, loss_hbm.at[wid])

        def p2(k, carry):
            base = k * _L
            valid = (base + lanes) < n_own
            sid = sid_v[pl.ds(base, _L)]
            posu = posu_v[pl.ds(base, _L)]
            sidr = jnp.where(valid, sid, _MAX_ITEMS + lanes)
            rank = jnp.zeros((_L,), jnp.int32)
            for j in range(_L - 1):
                sj = jnp.sum(jnp.where(lanes == j, sidr, 0))
                rank = rank + jnp.where((lanes > j) & (sidr == sj), 1, 0)
            maxrank = jnp.max(rank)
            cp_u = pltpu.async_copy(
                upd_hbm.at[posu], u_b.at[pl.ds(0, _L)], s0)

            def round_body(r):
                sel = (rank == r) & valid
                idx = jnp.where(sel, sid, _MAX_ITEMS)
                pltpu.async_copy(hist.at[idx], cur_v, s1).wait()

                def addc(c4, t):
                    for cc in range(4):
                        colv = lanes * 0 + (c4 * 4 + cc)
                        cu = plsc.load_gather(cur_v, [lanes, colv])
                        uu = plsc.load_gather(u_b, [lanes, colv])
                        plsc.store_scatter(
                            cur_v, [lanes, colv],
                            cu + jnp.where(sel, uu, 0.0))
                    return t

                lax.fori_loop(0, _D // 4, addc, jnp.int32(0))
                pltpu.async_copy(cur_v, hist.at[idx], s1).wait()
                return r + 1

            cp_u.wait()
            lax.while_loop(lambda r: r <= maxrank, round_body, jnp.int32(0))
            return carry

        lax.fori_loop(0, nchunks, p2, jnp.int32(0))

    return body(activations, samples, hist_ref)


def kernel(activations, samples, history, iterations):
    warm_up = _WARM_UP * iterations
    cool_down = _COOL_DOWN * iterations
    weight = _WEIGHT * warm_up / (1.0 + warm_up) / (1.0 + cool_down)

    hist_ref = jax.new_ref(history)
    loss_parts, _ = _sc_call(activations, samples, hist_ref)
    new_history = hist_ref[...]
    reg_loss = jnp.sum(loss_parts) * (weight / (_B * float(_D)))
    return activations, reg_loss, new_history, iterations + 1.0
